# Initial kernel scaffold; baseline (speedup 1.0000x reference)
import jax
import jax.numpy as jnp
from jax.experimental import pallas as pl


def kernel(X, A, H, W_r, W_u, W_c, b_r, b_u, b_c):
    raise NotImplementedError("write your pallas kernel here")


# speedup vs baseline: 1.4599x; 1.4599x over previous
import functools

import jax
import jax.numpy as jnp
from jax.experimental import pallas as pl
from jax.experimental.pallas import tpu as pltpu

B = 2
N = 2048
FX = 64
FH = 64
K = 5
FIN = FX + FH


def _gru_body(A_ref, X_ref, H_ref, Wru_ref, Wc_ref, br_ref, bu_ref, bc_ref,
              out_ref):
    A = A_ref[0]
    X = X_ref[0]
    H = H_ref[0]
    XH = jnp.concatenate([X, H], axis=-1)

    Pall = jnp.dot(XH, Wru_ref[...], preferred_element_type=jnp.float32)
    P = Pall[:, (K - 1) * 2 * FH:]
    for k in range(K - 2, -1, -1):
        P = (jnp.dot(A, P, preferred_element_type=jnp.float32)
             + Pall[:, k * 2 * FH:(k + 1) * 2 * FH])
    gate_r = jax.nn.sigmoid(P[:, :FH] + br_ref[...])
    gate_u = jax.nn.sigmoid(P[:, FH:] + bu_ref[...])

    XHr = jnp.concatenate([X, gate_r * H], axis=-1)
    Qall = jnp.dot(XHr, Wc_ref[...], preferred_element_type=jnp.float32)
    Q = Qall[:, (K - 1) * FH:]
    for k in range(K - 2, -1, -1):
        Q = (jnp.dot(A, Q, preferred_element_type=jnp.float32)
             + Qall[:, k * FH:(k + 1) * FH])
    cell = jnp.tanh(Q + bc_ref[...])
    out_ref[0] = gate_u * H + (1.0 - gate_u) * cell


@jax.jit
def kernel(X, A, H, W_r, W_u, W_c, b_r, b_u, b_c):
    Wru = jnp.concatenate([W_r, W_u], axis=-1)
    Wru = jnp.transpose(Wru, (1, 0, 2)).reshape(FIN, K * 2 * FH)
    Wc = jnp.transpose(W_c, (1, 0, 2)).reshape(FIN, K * FH)

    grid = (B,)
    out = pl.pallas_call(
        _gru_body,
        grid=grid,
        in_specs=[
            pl.BlockSpec((1, N, N), lambda b: (b, 0, 0)),
            pl.BlockSpec((1, N, FX), lambda b: (b, 0, 0)),
            pl.BlockSpec((1, N, FH), lambda b: (b, 0, 0)),
            pl.BlockSpec((FIN, K * 2 * FH), lambda b: (0, 0)),
            pl.BlockSpec((FIN, K * FH), lambda b: (0, 0)),
            pl.BlockSpec((N, FH), lambda b: (0, 0)),
            pl.BlockSpec((N, FH), lambda b: (0, 0)),
            pl.BlockSpec((N, FH), lambda b: (0, 0)),
        ],
        out_specs=pl.BlockSpec((1, N, FH), lambda b: (b, 0, 0)),
        out_shape=jax.ShapeDtypeStruct((B, N, FH), jnp.float32),
        compiler_params=pltpu.CompilerParams(
            dimension_semantics=("arbitrary",),
        ),
    )(A, X, H, Wru, Wc, b_r, b_u, b_c)
    return out
